# Initial kernel scaffold; baseline (speedup 1.0000x reference)
import jax
import jax.numpy as jnp
from jax.experimental import pallas as pl


def kernel(x, pos, edge_index, batch, sigma, We1, We2, Wh, Wx):
    raise NotImplementedError("write your pallas kernel here")


# speedup vs baseline: 3.9058x; 3.9058x over previous
import functools

import jax
import jax.numpy as jnp
from jax import lax
from jax.experimental import pallas as pl
from jax.experimental.pallas import tpu as pltpu
from jax.experimental.pallas import tpu_sc as plsc

N = 10000
E = 320000
D = 128
B = 64
SIGMA_DATA = 0.5

NW = 32
EPW = E // NW
K = 80
NCHUNK = EPW // K
NTILES = 16
NPAD = 10240
RPT = NPAD // NTILES
GPC = K // 16

UPT = 2560
UPAD = UPT * NTILES

NB = 1000
EB = 8000
F32 = jnp.float32
I32 = jnp.int32


def _p1_body(x_ref, pos1_ref, sig_ref, wa_ref, wca_ref, wb_ref, wcb_ref,
             psel_ref, a_ref, b_ref, pp_ref, npp_ref, pos4_ref):
    sig = sig_ref[...]
    c_in = lax.rsqrt(SIGMA_DATA * SIGMA_DATA + sig * sig)
    c_noise = jnp.log(sig) * 0.25
    cx = c_in * x_ref[...]
    a = (jnp.dot(cx, wa_ref[...], preferred_element_type=F32)
         + c_noise * wca_ref[...])
    b = (jnp.dot(cx, wb_ref[...], preferred_element_type=F32)
         + c_noise * wcb_ref[...])
    p1 = pos1_ref[...]
    cp = c_in * p1
    posp = jnp.dot(cp, psel_ref[...], preferred_element_type=F32)
    a_ref[...] = a
    b_ref[...] = b
    pp_ref[...] = posp
    npp_ref[...] = -posp
    lane = lax.broadcasted_iota(I32, p1.shape, 1)
    pos4_ref[...] = p1 * jnp.where(lane < 3, c_in, 1.0)


def _phase1(x, pos1, sig1, wa, wca, wb, wcb, psel):
    g = N // NB
    return pl.pallas_call(
        _p1_body,
        grid=(g,),
        in_specs=[
            pl.BlockSpec((NB, D), lambda i: (i, 0)),
            pl.BlockSpec((NB, 4), lambda i: (i, 0)),
            pl.BlockSpec((NB, 1), lambda i: (i, 0)),
            pl.BlockSpec((D, D), lambda i: (0, 0)),
            pl.BlockSpec((1, D), lambda i: (0, 0)),
            pl.BlockSpec((D, D), lambda i: (0, 0)),
            pl.BlockSpec((1, D), lambda i: (0, 0)),
            pl.BlockSpec((4, D), lambda i: (0, 0)),
        ],
        out_specs=[
            pl.BlockSpec((NB, D), lambda i: (i, 0)),
            pl.BlockSpec((NB, D), lambda i: (i, 0)),
            pl.BlockSpec((NB, D), lambda i: (i, 0)),
            pl.BlockSpec((NB, D), lambda i: (i, 0)),
            pl.BlockSpec((NB, 4), lambda i: (i, 0)),
        ],
        out_shape=[
            jax.ShapeDtypeStruct((N, D), F32),
            jax.ShapeDtypeStruct((N, D), F32),
            jax.ShapeDtypeStruct((N, D), F32),
            jax.ShapeDtypeStruct((N, D), F32),
            jax.ShapeDtypeStruct((N, 4), F32),
        ],
    )(x, pos1, sig1, wa, wca, wb, wcb, psel)


def _sc_gather_body(a_hbm, b_hbm, pp_hbm, npp_hbm, src_hbm, dst_hbm,
                    gsum_hbm, relp_hbm,
                    sidx, didx, gbuf, pbuf, sem1, sem2):
    c = lax.axis_index("c")
    s = lax.axis_index("s")
    wid = s * 2 + c
    base0 = wid * EPW
    pltpu.sync_copy(src_hbm.at[pl.ds(base0, EPW)], sidx)
    pltpu.sync_copy(dst_hbm.at[pl.ds(base0, EPW)], didx)

    def body(j, carry):
        off = j * K
        si = sidx.at[pl.ds(off, K)]
        di = didx.at[pl.ds(off, K)]
        ga = pltpu.async_copy(a_hbm.at[si], gbuf, sem1)
        gp = pltpu.async_copy(pp_hbm.at[si], pbuf, sem2)
        ga.wait()
        gb = pltpu.async_copy(b_hbm.at[di], gbuf, sem1, add=True)
        gp.wait()
        gn = pltpu.async_copy(npp_hbm.at[di], pbuf, sem2, add=True)
        gb.wait()
        pltpu.sync_copy(gbuf, gsum_hbm.at[pl.ds(base0 + off, K)])
        gn.wait()
        pltpu.sync_copy(pbuf, relp_hbm.at[pl.ds(base0 + off, K)])
        return carry

    lax.fori_loop(0, NCHUNK, body, 0)


def _phase2(a, b, pp, npp, src, dst):
    mesh = plsc.VectorSubcoreMesh(core_axis_name="c", subcore_axis_name="s")
    fn = pl.kernel(
        _sc_gather_body,
        out_type=(jax.ShapeDtypeStruct((E, D), F32),
                  jax.ShapeDtypeStruct((E, D), F32)),
        mesh=mesh,
        scratch_types=[
            pltpu.VMEM((EPW,), I32),
            pltpu.VMEM((EPW,), I32),
            pltpu.VMEM((K, D), F32),
            pltpu.VMEM((K, D), F32),
            pltpu.SemaphoreType.DMA,
            pltpu.SemaphoreType.DMA,
        ],
    )
    return fn(a, b, pp, npp, src, dst)


def _p3_body(gsum_ref, relp_ref, we2_ref, wxr_ref, we1d_ref,
             m2_ref, wx_ref, wy_ref, wz_ref):
    rp = relp_ref[...]
    rel = rp[:, 0:3]
    d2 = jnp.sum(rel * rel, axis=1, keepdims=True)
    pre = gsum_ref[...] + d2 * we1d_ref[...]
    m1 = pre * jax.nn.sigmoid(pre)
    m2 = jnp.dot(m1, we2_ref[...], preferred_element_type=F32)
    m2 = m2 * jax.nn.sigmoid(m2)
    m2_ref[...] = m2
    coef = jnp.tanh(jnp.sum(m2 * wxr_ref[...], axis=1, keepdims=True))
    wx_ref[...] = rp[:, 0:1] * coef
    wy_ref[...] = rp[:, 1:2] * coef
    wz_ref[...] = rp[:, 2:3] * coef


def _phase3(gsum, relp, we2, wxr, we1d):
    g = E // EB
    wspec = pl.BlockSpec((EB, 1), lambda i: (i, 0))
    return pl.pallas_call(
        _p3_body,
        grid=(g,),
        in_specs=[
            pl.BlockSpec((EB, D), lambda i: (i, 0)),
            pl.BlockSpec((EB, D), lambda i: (i, 0)),
            pl.BlockSpec((D, D), lambda i: (0, 0)),
            pl.BlockSpec((1, D), lambda i: (0, 0)),
            pl.BlockSpec((1, D), lambda i: (0, 0)),
        ],
        out_specs=[
            pl.BlockSpec((EB, D), lambda i: (i, 0)),
            wspec, wspec, wspec,
        ],
        out_shape=[
            jax.ShapeDtypeStruct((E, D), F32),
            jax.ShapeDtypeStruct((E, 1), F32),
            jax.ShapeDtypeStruct((E, 1), F32),
            jax.ShapeDtypeStruct((E, 1), F32),
        ],
    )(gsum, relp, we2, wxr, we1d)


def _sc_scatter_body(m2_hbm, wx_hbm, wy_hbm, wz_hbm, dst_hbm, z2d_hbm, z1d_hbm,
                     aggp_hbm, updp_hbm,
                     dbuf, mbuf, wxb, wyb, wzb, onesb,
                     ib0, ib1, ib2, ib3, stage, agg_sh, upd_sh):
    c = lax.axis_index("c")
    s = lax.axis_index("s")
    wid = s * 2 + c
    r0 = s * RPT
    u0 = s * UPT
    pltpu.sync_copy(z2d_hbm, agg_sh.at[pl.ds(r0, RPT)])
    pltpu.sync_copy(z1d_hbm, stage)
    pltpu.sync_copy(stage, upd_sh.at[pl.ds(u0, UPT)])
    for g in range(GPC):
        onesb[pl.ds(g * 16, 16)] = jnp.full((16,), 1.0, F32)
    plsc.subcore_barrier()
    ibs = (ib0, ib1, ib2, ib3)
    wbs = (wxb, wyb, wzb, onesb)

    def body(j, carry):
        base = wid * EPW + j * K
        pltpu.sync_copy(dst_hbm.at[pl.ds(base, K)], dbuf)
        pltpu.sync_copy(m2_hbm.at[pl.ds(base, K)], mbuf)
        pltpu.sync_copy(wx_hbm.at[pl.ds(base, K)], wxb)
        pltpu.sync_copy(wy_hbm.at[pl.ds(base, K)], wyb)
        pltpu.sync_copy(wz_hbm.at[pl.ds(base, K)], wzb)
        for g in range(GPC):
            d16 = dbuf[pl.ds(g * 16, 16)] * 4
            for jj in range(4):
                ibs[jj][pl.ds(g * 16, 16)] = d16 + jj
        pltpu.sync_copy(mbuf, agg_sh.at[dbuf], add=True)
        for jj in range(4):
            pltpu.sync_copy(wbs[jj], upd_sh.at[ibs[jj]], add=True)
        return carry

    lax.fori_loop(0, NCHUNK, body, 0)
    plsc.subcore_barrier()
    pltpu.sync_copy(agg_sh.at[pl.ds(r0, RPT)],
                    aggp_hbm.at[c].at[pl.ds(r0, RPT)])
    pltpu.sync_copy(upd_sh.at[pl.ds(u0, UPT)], stage)
    pltpu.sync_copy(stage, updp_hbm.at[c].at[pl.ds(u0, UPT)])


def _phase4(m2, wx, wy, wz, dst, z2d, z1d):
    mesh = plsc.VectorSubcoreMesh(core_axis_name="c", subcore_axis_name="s")
    fn = pl.kernel(
        _sc_scatter_body,
        out_type=(jax.ShapeDtypeStruct((2, NPAD, D), F32),
                  jax.ShapeDtypeStruct((2, UPAD), F32)),
        mesh=mesh,
        scratch_types=[
            pltpu.VMEM((K,), I32),
            pltpu.VMEM((K, D), F32),
            pltpu.VMEM((K,), F32),
            pltpu.VMEM((K,), F32),
            pltpu.VMEM((K,), F32),
            pltpu.VMEM((K,), F32),
            pltpu.VMEM((K,), I32),
            pltpu.VMEM((K,), I32),
            pltpu.VMEM((K,), I32),
            pltpu.VMEM((K,), I32),
            pltpu.VMEM((UPT,), F32),
            pltpu.VMEM_SHARED((NPAD, D), F32),
            pltpu.VMEM_SHARED((UPAD,), F32),
        ],
    )
    return fn(m2, wx, wy, wz, dst, z2d, z1d)


def _p5a_body(x_ref, sig_ref, batch_ref, aggp_ref, updp_ref,
              whx_ref, whc_ref, wha_ref,
              dx_ref, u4_ref, sums_ref):
    i = pl.program_id(0)
    sig = sig_ref[...]
    c_in = lax.rsqrt(SIGMA_DATA * SIGMA_DATA + sig * sig)
    c_noise = jnp.log(sig) * 0.25
    c_skip = (SIGMA_DATA * SIGMA_DATA) * c_in * c_in
    c_out = sig * SIGMA_DATA * c_in
    x = x_ref[...]
    cx = c_in * x
    agg = aggp_ref[0] + aggp_ref[1]
    u8 = updp_ref[0] + updp_ref[1]
    cnt = u8[:, 3:4]
    cnt = jnp.where(cnt == 0.0, 1.0, cnt)
    u3 = u8[:, 0:3] / cnt
    dx = (jnp.dot(cx, whx_ref[...], preferred_element_type=F32)
          + c_noise * whc_ref[...]
          + jnp.dot(agg, wha_ref[...], preferred_element_type=F32))
    dx_ref[...] = c_skip * x + c_out * (cx - dx)
    zero1 = jnp.zeros_like(cnt)
    u4_ref[...] = jnp.concatenate([u3, zero1], axis=1)
    bids = batch_ref[...]
    onehot = (bids == lax.broadcasted_iota(I32, (bids.shape[0], B), 1)
              ).astype(F32)
    u4c = jnp.concatenate([u3, jnp.ones_like(cnt)], axis=1)
    part = lax.dot_general(onehot, u4c, (((0,), (0,)), ((), ())),
                           preferred_element_type=F32)

    @pl.when(i == 0)
    def _():
        sums_ref[...] = part

    @pl.when(i > 0)
    def _():
        sums_ref[...] += part


def _phase5a(x, sig1, batch1, aggp, updp4, whx, whc, wha):
    g = N // NB
    return pl.pallas_call(
        _p5a_body,
        grid=(g,),
        in_specs=[
            pl.BlockSpec((NB, D), lambda i: (i, 0)),
            pl.BlockSpec((NB, 1), lambda i: (i, 0)),
            pl.BlockSpec((NB, 1), lambda i: (i, 0)),
            pl.BlockSpec((2, NB, D), lambda i: (0, i, 0)),
            pl.BlockSpec((2, NB, 4), lambda i: (0, i, 0)),
            pl.BlockSpec((D, D), lambda i: (0, 0)),
            pl.BlockSpec((1, D), lambda i: (0, 0)),
            pl.BlockSpec((D, D), lambda i: (0, 0)),
        ],
        out_specs=[
            pl.BlockSpec((NB, D), lambda i: (i, 0)),
            pl.BlockSpec((NB, 4), lambda i: (i, 0)),
            pl.BlockSpec((B, 4), lambda i: (0, 0)),
        ],
        out_shape=[
            jax.ShapeDtypeStruct((N, D), F32),
            jax.ShapeDtypeStruct((N, 4), F32),
            jax.ShapeDtypeStruct((B, 4), F32),
        ],
    )(x, sig1, batch1, aggp, updp4, whx, whc, wha)


def _p5b_body(rpos4_ref, pos4_ref, sig_ref, batch_ref, u4_ref, sums_ref,
              dpos_ref):
    sig = sig_ref[...]
    c_in = lax.rsqrt(SIGMA_DATA * SIGMA_DATA + sig * sig)
    c_skip = (SIGMA_DATA * SIGMA_DATA) * c_in * c_in
    c_out = sig * SIGMA_DATA * c_in
    sums = sums_ref[...]
    cntb = sums[:, 3:4]
    cntb = jnp.where(cntb == 0.0, 1.0, cntb)
    means4 = sums / cntb
    bids = batch_ref[...]
    onehot = (bids == lax.broadcasted_iota(I32, (bids.shape[0], B), 1)
              ).astype(F32)
    mrow = jnp.dot(onehot, means4, preferred_element_type=F32)
    fpos = pos4_ref[...] + (u4_ref[...] - mrow)
    res = c_skip * rpos4_ref[...] + c_out * fpos
    dpos_ref[...] = res[:, 0:3]


def _phase5b(rpos4, pos4, sig1, batch1, u4, sums):
    g = N // NB
    return pl.pallas_call(
        _p5b_body,
        grid=(g,),
        in_specs=[
            pl.BlockSpec((NB, 4), lambda i: (i, 0)),
            pl.BlockSpec((NB, 4), lambda i: (i, 0)),
            pl.BlockSpec((NB, 1), lambda i: (i, 0)),
            pl.BlockSpec((NB, 1), lambda i: (i, 0)),
            pl.BlockSpec((NB, 4), lambda i: (i, 0)),
            pl.BlockSpec((B, 4), lambda i: (0, 0)),
        ],
        out_specs=pl.BlockSpec((NB, 3), lambda i: (i, 0)),
        out_shape=jax.ShapeDtypeStruct((N, 3), F32),
    )(rpos4, pos4, sig1, batch1, u4, sums)


def kernel(x, pos, edge_index, batch, sigma, We1, We2, Wh, Wx):
    src = edge_index[0]
    dst = edge_index[1]
    sig1 = sigma.reshape(N, 1)
    batch1 = batch.reshape(N, 1)
    pos1 = jnp.concatenate([pos, jnp.ones((N, 1), F32)], axis=1)
    rpos4 = jnp.concatenate([pos, jnp.zeros((N, 1), F32)], axis=1)
    wa = We1[0:D]
    wca = We1[D:D + 1]
    wb = We1[D + 1:2 * D + 1]
    wcb = We1[2 * D + 1:2 * D + 2]
    we1d = We1[2 * D + 2:2 * D + 3]
    whx = Wh[0:D]
    whc = Wh[D:D + 1]
    wha = Wh[D + 1:2 * D + 1]
    wxr = Wx.reshape(1, D)
    psel = jnp.where((jnp.arange(4, dtype=I32)[:, None]
                      == jnp.arange(D, dtype=I32)[None, :])
                     & (jnp.arange(4, dtype=I32)[:, None] < 3),
                     jnp.float32(1.0), jnp.float32(0.0))
    z2d = jnp.zeros((RPT, D), F32)
    z1d = jnp.zeros((UPT,), F32)

    a, b, pp, npp, pos4 = _phase1(x, pos1, sig1, wa, wca, wb, wcb, psel)
    gsum, relp = _phase2(a, b, pp, npp, src, dst)
    m2, wx3, wy3, wz3 = _phase3(gsum, relp, We2, wxr, we1d)
    aggp, updp = _phase4(m2, wx3.reshape(E), wy3.reshape(E), wz3.reshape(E),
                         dst, z2d, z1d)
    aggp = aggp[:, :N]
    updp4 = updp[:, :4 * N].reshape(2, N, 4)
    d_x, u4, sums = _phase5a(x, sig1, batch1, aggp, updp4, whx, whc, wha)
    d_pos = _phase5b(rpos4, pos4, sig1, batch1, u4, sums)
    return (d_x, d_pos)


# speedup vs baseline: 4.8738x; 1.2479x over previous
import functools

import jax
import jax.numpy as jnp
from jax import lax
from jax.experimental import pallas as pl
from jax.experimental.pallas import tpu as pltpu
from jax.experimental.pallas import tpu_sc as plsc

N = 10000
E = 320000
D = 128
B = 64
SIGMA_DATA = 0.5

NW = 32
EPW = E // NW
K = 80
NCHUNK = EPW // K
NTILES = 16
NPAD = 10240
RPT = NPAD // NTILES
GPC = K // 16

UPT = 2560
UPAD = UPT * NTILES

NB = 1000
EB = 8000
F32 = jnp.float32
I32 = jnp.int32


def _p1_body(x_ref, pos1_ref, sig_ref, wa_ref, wca_ref, wb_ref, wcb_ref,
             psel_ref, a_ref, b_ref, pp_ref, npp_ref, pos4_ref):
    sig = sig_ref[...]
    c_in = lax.rsqrt(SIGMA_DATA * SIGMA_DATA + sig * sig)
    c_noise = jnp.log(sig) * 0.25
    cx = c_in * x_ref[...]
    a = (jnp.dot(cx, wa_ref[...], preferred_element_type=F32)
         + c_noise * wca_ref[...])
    b = (jnp.dot(cx, wb_ref[...], preferred_element_type=F32)
         + c_noise * wcb_ref[...])
    p1 = pos1_ref[...]
    cp = c_in * p1
    posp = jnp.dot(cp, psel_ref[...], preferred_element_type=F32)
    a_ref[...] = a
    b_ref[...] = b
    pp_ref[...] = posp
    npp_ref[...] = -posp
    lane = lax.broadcasted_iota(I32, p1.shape, 1)
    pos4_ref[...] = p1 * jnp.where(lane < 3, c_in, 1.0)


def _phase1(x, pos1, sig1, wa, wca, wb, wcb, psel):
    g = N // NB
    return pl.pallas_call(
        _p1_body,
        grid=(g,),
        in_specs=[
            pl.BlockSpec((NB, D), lambda i: (i, 0)),
            pl.BlockSpec((NB, 4), lambda i: (i, 0)),
            pl.BlockSpec((NB, 1), lambda i: (i, 0)),
            pl.BlockSpec((D, D), lambda i: (0, 0)),
            pl.BlockSpec((1, D), lambda i: (0, 0)),
            pl.BlockSpec((D, D), lambda i: (0, 0)),
            pl.BlockSpec((1, D), lambda i: (0, 0)),
            pl.BlockSpec((4, D), lambda i: (0, 0)),
        ],
        out_specs=[
            pl.BlockSpec((NB, D), lambda i: (i, 0)),
            pl.BlockSpec((NB, D), lambda i: (i, 0)),
            pl.BlockSpec((NB, D), lambda i: (i, 0)),
            pl.BlockSpec((NB, D), lambda i: (i, 0)),
            pl.BlockSpec((NB, 4), lambda i: (i, 0)),
        ],
        out_shape=[
            jax.ShapeDtypeStruct((N, D), F32),
            jax.ShapeDtypeStruct((N, D), F32),
            jax.ShapeDtypeStruct((N, D), F32),
            jax.ShapeDtypeStruct((N, D), F32),
            jax.ShapeDtypeStruct((N, 4), F32),
        ],
    )(x, pos1, sig1, wa, wca, wb, wcb, psel)


def _sc_gather_body(a_hbm, b_hbm, pp_hbm, npp_hbm, src_hbm, dst_hbm,
                    gsum_hbm, relp_hbm,
                    sidx, didx, gbuf, pbuf, sem1, sem2):
    c = lax.axis_index("c")
    s = lax.axis_index("s")
    wid = s * 2 + c
    base0 = wid * EPW
    pltpu.sync_copy(src_hbm.at[pl.ds(base0, EPW)], sidx)
    pltpu.sync_copy(dst_hbm.at[pl.ds(base0, EPW)], didx)

    def body(j, carry):
        off = j * K
        si = sidx.at[pl.ds(off, K)]
        di = didx.at[pl.ds(off, K)]
        ga = pltpu.async_copy(a_hbm.at[si], gbuf, sem1)
        gp = pltpu.async_copy(pp_hbm.at[si], pbuf, sem2)
        ga.wait()
        gb = pltpu.async_copy(b_hbm.at[di], gbuf, sem1, add=True)
        gp.wait()
        gn = pltpu.async_copy(npp_hbm.at[di], pbuf, sem2, add=True)
        gb.wait()
        pltpu.sync_copy(gbuf, gsum_hbm.at[pl.ds(base0 + off, K)])
        gn.wait()
        pltpu.sync_copy(pbuf, relp_hbm.at[pl.ds(base0 + off, K)])
        return carry

    lax.fori_loop(0, NCHUNK, body, 0)


def _phase2(a, b, pp, npp, src, dst):
    mesh = plsc.VectorSubcoreMesh(core_axis_name="c", subcore_axis_name="s")
    fn = pl.kernel(
        _sc_gather_body,
        out_type=(jax.ShapeDtypeStruct((E, D), F32),
                  jax.ShapeDtypeStruct((E, D), F32)),
        mesh=mesh,
        scratch_types=[
            pltpu.VMEM((EPW,), I32),
            pltpu.VMEM((EPW,), I32),
            pltpu.VMEM((K, D), F32),
            pltpu.VMEM((K, D), F32),
            pltpu.SemaphoreType.DMA,
            pltpu.SemaphoreType.DMA,
        ],
    )
    return fn(a, b, pp, npp, src, dst)


def _p3_body(gsum_ref, relp_ref, we2_ref, wxr_ref, we1d_ref,
             m2_ref, wx_ref, wy_ref, wz_ref):
    rp = relp_ref[...]
    rel = rp[:, 0:3]
    d2 = jnp.sum(rel * rel, axis=1, keepdims=True)
    pre = gsum_ref[...] + d2 * we1d_ref[...]
    m1 = pre * jax.nn.sigmoid(pre)
    m2 = jnp.dot(m1, we2_ref[...], preferred_element_type=F32)
    m2 = m2 * jax.nn.sigmoid(m2)
    m2_ref[...] = m2
    coef = jnp.tanh(jnp.sum(m2 * wxr_ref[...], axis=1, keepdims=True))
    wx_ref[...] = rp[:, 0:1] * coef
    wy_ref[...] = rp[:, 1:2] * coef
    wz_ref[...] = rp[:, 2:3] * coef


def _phase3(gsum, relp, we2, wxr, we1d):
    g = E // EB
    wspec = pl.BlockSpec((EB, 1), lambda i: (i, 0))
    return pl.pallas_call(
        _p3_body,
        grid=(g,),
        in_specs=[
            pl.BlockSpec((EB, D), lambda i: (i, 0)),
            pl.BlockSpec((EB, D), lambda i: (i, 0)),
            pl.BlockSpec((D, D), lambda i: (0, 0)),
            pl.BlockSpec((1, D), lambda i: (0, 0)),
            pl.BlockSpec((1, D), lambda i: (0, 0)),
        ],
        out_specs=[
            pl.BlockSpec((EB, D), lambda i: (i, 0)),
            wspec, wspec, wspec,
        ],
        out_shape=[
            jax.ShapeDtypeStruct((E, D), F32),
            jax.ShapeDtypeStruct((E, 1), F32),
            jax.ShapeDtypeStruct((E, 1), F32),
            jax.ShapeDtypeStruct((E, 1), F32),
        ],
    )(gsum, relp, we2, wxr, we1d)


def _sc_scatter_body(m2_hbm, wx_hbm, wy_hbm, wz_hbm, dst_hbm, z2d_hbm, z1d_hbm,
                     aggp_hbm, updp_hbm,
                     dbuf0, mbuf0, wxb0, wyb0, wzb0, ib00, ib01, ib02, ib03,
                     dbuf1, mbuf1, wxb1, wyb1, wzb1, ib10, ib11, ib12, ib13,
                     onesb, stage, agg_sh, upd_sh,
                     lsem0, lsem1, ssem0, ssem1):
    c = lax.axis_index("c")
    s = lax.axis_index("s")
    wid = s * 2 + c
    r0 = s * RPT
    u0 = s * UPT
    base0 = wid * EPW
    pltpu.sync_copy(z2d_hbm, agg_sh.at[pl.ds(r0, RPT)])
    pltpu.sync_copy(z1d_hbm, stage)
    pltpu.sync_copy(stage, upd_sh.at[pl.ds(u0, UPT)])
    for g in range(GPC):
        onesb[pl.ds(g * 16, 16)] = jnp.full((16,), 1.0, F32)
    plsc.subcore_barrier()

    dbufs = (dbuf0, dbuf1)
    mbufs = (mbuf0, mbuf1)
    wxbs = (wxb0, wxb1)
    wybs = (wyb0, wyb1)
    wzbs = (wzb0, wzb1)
    ibss = ((ib00, ib01, ib02, ib03), (ib10, ib11, ib12, ib13))
    lsems = (lsem0, lsem1)
    ssems = (ssem0, ssem1)

    def fire_loads(jj, b):
        base = base0 + jj * K
        pltpu.async_copy(dst_hbm.at[pl.ds(base, K)], dbufs[b], lsems[b])
        pltpu.async_copy(m2_hbm.at[pl.ds(base, K)], mbufs[b], lsems[b])
        pltpu.async_copy(wx_hbm.at[pl.ds(base, K)], wxbs[b], lsems[b])
        pltpu.async_copy(wy_hbm.at[pl.ds(base, K)], wybs[b], lsems[b])
        pltpu.async_copy(wz_hbm.at[pl.ds(base, K)], wzbs[b], lsems[b])

    def wait_loads(b):
        base = base0
        pltpu.make_async_copy(dst_hbm.at[pl.ds(base, K)], dbufs[b],
                              lsems[b]).wait()
        pltpu.make_async_copy(m2_hbm.at[pl.ds(base, K)], mbufs[b],
                              lsems[b]).wait()
        pltpu.make_async_copy(wx_hbm.at[pl.ds(base, K)], wxbs[b],
                              lsems[b]).wait()
        pltpu.make_async_copy(wy_hbm.at[pl.ds(base, K)], wybs[b],
                              lsems[b]).wait()
        pltpu.make_async_copy(wz_hbm.at[pl.ds(base, K)], wzbs[b],
                              lsems[b]).wait()

    def fire_scatters(b):
        ibs = ibss[b]
        wbs = (wxbs[b], wybs[b], wzbs[b], onesb)
        for g in range(GPC):
            d16 = dbufs[b][pl.ds(g * 16, 16)] * 4
            for jj in range(4):
                ibs[jj][pl.ds(g * 16, 16)] = d16 + jj
        pltpu.async_copy(mbufs[b], agg_sh.at[dbufs[b]], ssems[b], add=True)
        for jj in range(4):
            pltpu.async_copy(wbs[jj], upd_sh.at[ibs[jj]], ssems[b], add=True)

    def wait_scatters(b):
        ibs = ibss[b]
        wbs = (wxbs[b], wybs[b], wzbs[b], onesb)
        pltpu.make_async_copy(mbufs[b], agg_sh.at[dbufs[b]], ssems[b]).wait()
        for jj in range(4):
            pltpu.make_async_copy(wbs[jj], upd_sh.at[ibs[jj]],
                                  ssems[b]).wait()

    fire_loads(0, 0)

    def body(t, carry):
        j0 = t * 2
        wait_loads(0)

        @pl.when(t > 0)
        def _():
            wait_scatters(1)

        fire_loads(j0 + 1, 1)
        fire_scatters(0)
        wait_loads(1)
        wait_scatters(0)

        @pl.when(t < NCHUNK // 2 - 1)
        def _():
            fire_loads(j0 + 2, 0)

        fire_scatters(1)
        return carry

    lax.fori_loop(0, NCHUNK // 2, body, 0)
    wait_scatters(1)
    fire_loads(NCHUNK - 1, 0)
    wait_loads(0)
    fire_scatters(0)
    wait_scatters(0)
    plsc.subcore_barrier()
    pltpu.sync_copy(agg_sh.at[pl.ds(r0, RPT)],
                    aggp_hbm.at[c].at[pl.ds(r0, RPT)])
    pltpu.sync_copy(upd_sh.at[pl.ds(u0, UPT)], stage)
    pltpu.sync_copy(stage, updp_hbm.at[c].at[pl.ds(u0, UPT)])


def _phase4(m2, wx, wy, wz, dst, z2d, z1d):
    mesh = plsc.VectorSubcoreMesh(core_axis_name="c", subcore_axis_name="s")
    kb = [
        pltpu.VMEM((K,), I32),
        pltpu.VMEM((K, D), F32),
        pltpu.VMEM((K,), F32),
        pltpu.VMEM((K,), F32),
        pltpu.VMEM((K,), F32),
        pltpu.VMEM((K,), I32),
        pltpu.VMEM((K,), I32),
        pltpu.VMEM((K,), I32),
        pltpu.VMEM((K,), I32),
    ]
    fn = pl.kernel(
        _sc_scatter_body,
        out_type=(jax.ShapeDtypeStruct((2, NPAD, D), F32),
                  jax.ShapeDtypeStruct((2, UPAD), F32)),
        mesh=mesh,
        scratch_types=kb + kb + [
            pltpu.VMEM((K,), F32),
            pltpu.VMEM((UPT,), F32),
            pltpu.VMEM_SHARED((NPAD, D), F32),
            pltpu.VMEM_SHARED((UPAD,), F32),
            pltpu.SemaphoreType.DMA,
            pltpu.SemaphoreType.DMA,
            pltpu.SemaphoreType.DMA,
            pltpu.SemaphoreType.DMA,
        ],
    )
    return fn(m2, wx, wy, wz, dst, z2d, z1d)


def _p5a_body(x_ref, sig_ref, batch_ref, aggp_ref, updp_ref,
              whx_ref, whc_ref, wha_ref,
              dx_ref, u4_ref, sums_ref):
    i = pl.program_id(0)
    sig = sig_ref[...]
    c_in = lax.rsqrt(SIGMA_DATA * SIGMA_DATA + sig * sig)
    c_noise = jnp.log(sig) * 0.25
    c_skip = (SIGMA_DATA * SIGMA_DATA) * c_in * c_in
    c_out = sig * SIGMA_DATA * c_in
    x = x_ref[...]
    cx = c_in * x
    agg = aggp_ref[0] + aggp_ref[1]
    u8 = updp_ref[0] + updp_ref[1]
    cnt = u8[:, 3:4]
    cnt = jnp.where(cnt == 0.0, 1.0, cnt)
    u3 = u8[:, 0:3] / cnt
    dx = (jnp.dot(cx, whx_ref[...], preferred_element_type=F32)
          + c_noise * whc_ref[...]
          + jnp.dot(agg, wha_ref[...], preferred_element_type=F32))
    dx_ref[...] = c_skip * x + c_out * (cx - dx)
    zero1 = jnp.zeros_like(cnt)
    u4_ref[...] = jnp.concatenate([u3, zero1], axis=1)
    bids = batch_ref[...]
    onehot = (bids == lax.broadcasted_iota(I32, (bids.shape[0], B), 1)
              ).astype(F32)
    u4c = jnp.concatenate([u3, jnp.ones_like(cnt)], axis=1)
    part = lax.dot_general(onehot, u4c, (((0,), (0,)), ((), ())),
                           preferred_element_type=F32)

    @pl.when(i == 0)
    def _():
        sums_ref[...] = part

    @pl.when(i > 0)
    def _():
        sums_ref[...] += part


def _phase5a(x, sig1, batch1, aggp, updp4, whx, whc, wha):
    g = N // NB
    return pl.pallas_call(
        _p5a_body,
        grid=(g,),
        in_specs=[
            pl.BlockSpec((NB, D), lambda i: (i, 0)),
            pl.BlockSpec((NB, 1), lambda i: (i, 0)),
            pl.BlockSpec((NB, 1), lambda i: (i, 0)),
            pl.BlockSpec((2, NB, D), lambda i: (0, i, 0)),
            pl.BlockSpec((2, NB, 4), lambda i: (0, i, 0)),
            pl.BlockSpec((D, D), lambda i: (0, 0)),
            pl.BlockSpec((1, D), lambda i: (0, 0)),
            pl.BlockSpec((D, D), lambda i: (0, 0)),
        ],
        out_specs=[
            pl.BlockSpec((NB, D), lambda i: (i, 0)),
            pl.BlockSpec((NB, 4), lambda i: (i, 0)),
            pl.BlockSpec((B, 4), lambda i: (0, 0)),
        ],
        out_shape=[
            jax.ShapeDtypeStruct((N, D), F32),
            jax.ShapeDtypeStruct((N, 4), F32),
            jax.ShapeDtypeStruct((B, 4), F32),
        ],
    )(x, sig1, batch1, aggp, updp4, whx, whc, wha)


def _p5b_body(rpos4_ref, pos4_ref, sig_ref, batch_ref, u4_ref, sums_ref,
              dpos_ref):
    sig = sig_ref[...]
    c_in = lax.rsqrt(SIGMA_DATA * SIGMA_DATA + sig * sig)
    c_skip = (SIGMA_DATA * SIGMA_DATA) * c_in * c_in
    c_out = sig * SIGMA_DATA * c_in
    sums = sums_ref[...]
    cntb = sums[:, 3:4]
    cntb = jnp.where(cntb == 0.0, 1.0, cntb)
    means4 = sums / cntb
    bids = batch_ref[...]
    onehot = (bids == lax.broadcasted_iota(I32, (bids.shape[0], B), 1)
              ).astype(F32)
    mrow = jnp.dot(onehot, means4, preferred_element_type=F32)
    fpos = pos4_ref[...] + (u4_ref[...] - mrow)
    res = c_skip * rpos4_ref[...] + c_out * fpos
    dpos_ref[...] = res[:, 0:3]


def _phase5b(rpos4, pos4, sig1, batch1, u4, sums):
    g = N // NB
    return pl.pallas_call(
        _p5b_body,
        grid=(g,),
        in_specs=[
            pl.BlockSpec((NB, 4), lambda i: (i, 0)),
            pl.BlockSpec((NB, 4), lambda i: (i, 0)),
            pl.BlockSpec((NB, 1), lambda i: (i, 0)),
            pl.BlockSpec((NB, 1), lambda i: (i, 0)),
            pl.BlockSpec((NB, 4), lambda i: (i, 0)),
            pl.BlockSpec((B, 4), lambda i: (0, 0)),
        ],
        out_specs=pl.BlockSpec((NB, 3), lambda i: (i, 0)),
        out_shape=jax.ShapeDtypeStruct((N, 3), F32),
    )(rpos4, pos4, sig1, batch1, u4, sums)


def kernel(x, pos, edge_index, batch, sigma, We1, We2, Wh, Wx):
    src = edge_index[0]
    dst = edge_index[1]
    sig1 = sigma.reshape(N, 1)
    batch1 = batch.reshape(N, 1)
    pos1 = jnp.concatenate([pos, jnp.ones((N, 1), F32)], axis=1)
    rpos4 = jnp.concatenate([pos, jnp.zeros((N, 1), F32)], axis=1)
    wa = We1[0:D]
    wca = We1[D:D + 1]
    wb = We1[D + 1:2 * D + 1]
    wcb = We1[2 * D + 1:2 * D + 2]
    we1d = We1[2 * D + 2:2 * D + 3]
    whx = Wh[0:D]
    whc = Wh[D:D + 1]
    wha = Wh[D + 1:2 * D + 1]
    wxr = Wx.reshape(1, D)
    psel = jnp.where((jnp.arange(4, dtype=I32)[:, None]
                      == jnp.arange(D, dtype=I32)[None, :])
                     & (jnp.arange(4, dtype=I32)[:, None] < 3),
                     jnp.float32(1.0), jnp.float32(0.0))
    z2d = jnp.zeros((RPT, D), F32)
    z1d = jnp.zeros((UPT,), F32)

    a, b, pp, npp, pos4 = _phase1(x, pos1, sig1, wa, wca, wb, wcb, psel)
    gsum, relp = _phase2(a, b, pp, npp, src, dst)
    m2, wx3, wy3, wz3 = _phase3(gsum, relp, We2, wxr, we1d)
    aggp, updp = _phase4(m2, wx3.reshape(E), wy3.reshape(E), wz3.reshape(E),
                         dst, z2d, z1d)
    aggp = aggp[:, :N]
    updp4 = updp[:, :4 * N].reshape(2, N, 4)
    d_x, u4, sums = _phase5a(x, sig1, batch1, aggp, updp4, whx, whc, wha)
    d_pos = _phase5b(rpos4, pos4, sig1, batch1, u4, sums)
    return (d_x, d_pos)


# speedup vs baseline: 5.1722x; 1.0612x over previous
import functools

import jax
import jax.numpy as jnp
from jax import lax
from jax.experimental import pallas as pl
from jax.experimental.pallas import tpu as pltpu
from jax.experimental.pallas import tpu_sc as plsc

N = 10000
E = 320000
D = 128
B = 64
SIGMA_DATA = 0.5

NW = 32
EPW = E // NW
K = 80
NCHUNK = EPW // K
NTILES = 16
NPAD = 10240
RPT = NPAD // NTILES
GPC = K // 16

UPT = 2560
UPAD = UPT * NTILES

NB = 1000
EB = 8000
F32 = jnp.float32
I32 = jnp.int32


def _p1_body(x_ref, pos1_ref, sig_ref, wa_ref, wca_ref, wb_ref, wcb_ref,
             psel_ref, a_ref, b_ref, pp_ref, npp_ref, pos4_ref):
    sig = sig_ref[...]
    c_in = lax.rsqrt(SIGMA_DATA * SIGMA_DATA + sig * sig)
    c_noise = jnp.log(sig) * 0.25
    cx = c_in * x_ref[...]
    a = (jnp.dot(cx, wa_ref[...], preferred_element_type=F32)
         + c_noise * wca_ref[...])
    b = (jnp.dot(cx, wb_ref[...], preferred_element_type=F32)
         + c_noise * wcb_ref[...])
    p1 = pos1_ref[...]
    cp = c_in * p1
    posp = jnp.dot(cp, psel_ref[...], preferred_element_type=F32)
    a_ref[...] = a
    b_ref[...] = b
    pp_ref[...] = posp
    npp_ref[...] = -posp
    lane = lax.broadcasted_iota(I32, p1.shape, 1)
    pos4_ref[...] = p1 * jnp.where(lane < 3, c_in, 1.0)


def _phase1(x, pos1, sig1, wa, wca, wb, wcb, psel):
    g = N // NB
    return pl.pallas_call(
        _p1_body,
        grid=(g,),
        in_specs=[
            pl.BlockSpec((NB, D), lambda i: (i, 0)),
            pl.BlockSpec((NB, 4), lambda i: (i, 0)),
            pl.BlockSpec((NB, 1), lambda i: (i, 0)),
            pl.BlockSpec((D, D), lambda i: (0, 0)),
            pl.BlockSpec((1, D), lambda i: (0, 0)),
            pl.BlockSpec((D, D), lambda i: (0, 0)),
            pl.BlockSpec((1, D), lambda i: (0, 0)),
            pl.BlockSpec((4, D), lambda i: (0, 0)),
        ],
        out_specs=[
            pl.BlockSpec((NB, D), lambda i: (i, 0)),
            pl.BlockSpec((NB, D), lambda i: (i, 0)),
            pl.BlockSpec((NB, D), lambda i: (i, 0)),
            pl.BlockSpec((NB, D), lambda i: (i, 0)),
            pl.BlockSpec((NB, 4), lambda i: (i, 0)),
        ],
        out_shape=[
            jax.ShapeDtypeStruct((N, D), F32),
            jax.ShapeDtypeStruct((N, D), F32),
            jax.ShapeDtypeStruct((N, D), F32),
            jax.ShapeDtypeStruct((N, D), F32),
            jax.ShapeDtypeStruct((N, 4), F32),
        ],
    )(x, pos1, sig1, wa, wca, wb, wcb, psel)


def _sc_gather_body(a_hbm, b_hbm, pp_hbm, npp_hbm, src_hbm, dst_hbm,
                    gsum_hbm, relp_hbm,
                    sidx, didx, gbuf0, pbuf0, gbuf1, pbuf1,
                    gsem0, psem0, wsem0, gsem1, psem1, wsem1):
    c = lax.axis_index("c")
    s = lax.axis_index("s")
    wid = s * 2 + c
    base0 = wid * EPW
    pltpu.sync_copy(src_hbm.at[pl.ds(base0, EPW)], sidx)
    pltpu.sync_copy(dst_hbm.at[pl.ds(base0, EPW)], didx)
    gbufs = (gbuf0, gbuf1)
    pbufs = (pbuf0, pbuf1)
    gsems = (gsem0, gsem1)
    psems = (psem0, psem1)
    wsems = (wsem0, wsem1)

    def wait_writes(b):
        pltpu.make_async_copy(gbufs[b], gsum_hbm.at[pl.ds(base0, K)],
                              wsems[b]).wait()
        pltpu.make_async_copy(pbufs[b], relp_hbm.at[pl.ds(base0, K)],
                              wsems[b]).wait()

    def part(jj, b, first):
        off = jj * K
        si = sidx.at[pl.ds(off, K)]
        di = didx.at[pl.ds(off, K)]
        if not first:
            wait_writes(b)
        pltpu.async_copy(a_hbm.at[si], gbufs[b], gsems[b])
        pltpu.async_copy(pp_hbm.at[si], pbufs[b], psems[b])
        pltpu.make_async_copy(a_hbm.at[si], gbufs[b], gsems[b]).wait()
        pltpu.async_copy(b_hbm.at[di], gbufs[b], gsems[b], add=True)
        pltpu.make_async_copy(pp_hbm.at[si], pbufs[b], psems[b]).wait()
        pltpu.async_copy(npp_hbm.at[di], pbufs[b], psems[b], add=True)
        pltpu.make_async_copy(b_hbm.at[di], gbufs[b], gsems[b]).wait()
        pltpu.async_copy(gbufs[b], gsum_hbm.at[pl.ds(base0 + off, K)],
                         wsems[b])
        pltpu.make_async_copy(npp_hbm.at[di], pbufs[b], psems[b]).wait()
        pltpu.async_copy(pbufs[b], relp_hbm.at[pl.ds(base0 + off, K)],
                         wsems[b])

    def body(t, carry):
        @pl.when(t == 0)
        def _():
            part(0, 0, True)
            part(1, 1, True)

        @pl.when(t > 0)
        def _():
            part(2 * t, 0, False)
            part(2 * t + 1, 1, False)

        return carry

    lax.fori_loop(0, NCHUNK // 2, body, 0)
    wait_writes(0)
    part(NCHUNK - 1, 0, True)
    wait_writes(0)
    wait_writes(1)


def _phase2(a, b, pp, npp, src, dst):
    mesh = plsc.VectorSubcoreMesh(core_axis_name="c", subcore_axis_name="s")
    fn = pl.kernel(
        _sc_gather_body,
        out_type=(jax.ShapeDtypeStruct((E, D), F32),
                  jax.ShapeDtypeStruct((E, D), F32)),
        mesh=mesh,
        scratch_types=[
            pltpu.VMEM((EPW,), I32),
            pltpu.VMEM((EPW,), I32),
            pltpu.VMEM((K, D), F32),
            pltpu.VMEM((K, D), F32),
            pltpu.VMEM((K, D), F32),
            pltpu.VMEM((K, D), F32),
            pltpu.SemaphoreType.DMA,
            pltpu.SemaphoreType.DMA,
            pltpu.SemaphoreType.DMA,
            pltpu.SemaphoreType.DMA,
            pltpu.SemaphoreType.DMA,
            pltpu.SemaphoreType.DMA,
        ],
    )
    return fn(a, b, pp, npp, src, dst)


def _p3_body(gsum_ref, relp_ref, we2_ref, wxr_ref, we1d_ref,
             m2_ref, wx_ref, wy_ref, wz_ref):
    rp = relp_ref[...]
    rel = rp[:, 0:3]
    d2 = jnp.sum(rel * rel, axis=1, keepdims=True)
    pre = gsum_ref[...] + d2 * we1d_ref[...]
    m1 = pre * jax.nn.sigmoid(pre)
    m2 = jnp.dot(m1, we2_ref[...], preferred_element_type=F32)
    m2 = m2 * jax.nn.sigmoid(m2)
    m2_ref[...] = m2
    coef = jnp.tanh(jnp.sum(m2 * wxr_ref[...], axis=1, keepdims=True))
    wx_ref[...] = rp[:, 0:1] * coef
    wy_ref[...] = rp[:, 1:2] * coef
    wz_ref[...] = rp[:, 2:3] * coef


def _phase3(gsum, relp, we2, wxr, we1d):
    g = E // EB
    wspec = pl.BlockSpec((EB, 1), lambda i: (i, 0))
    return pl.pallas_call(
        _p3_body,
        grid=(g,),
        in_specs=[
            pl.BlockSpec((EB, D), lambda i: (i, 0)),
            pl.BlockSpec((EB, D), lambda i: (i, 0)),
            pl.BlockSpec((D, D), lambda i: (0, 0)),
            pl.BlockSpec((1, D), lambda i: (0, 0)),
            pl.BlockSpec((1, D), lambda i: (0, 0)),
        ],
        out_specs=[
            pl.BlockSpec((EB, D), lambda i: (i, 0)),
            wspec, wspec, wspec,
        ],
        out_shape=[
            jax.ShapeDtypeStruct((E, D), F32),
            jax.ShapeDtypeStruct((E, 1), F32),
            jax.ShapeDtypeStruct((E, 1), F32),
            jax.ShapeDtypeStruct((E, 1), F32),
        ],
    )(gsum, relp, we2, wxr, we1d)


def _sc_scatter_body(m2_hbm, wx_hbm, wy_hbm, wz_hbm, dst_hbm, z2d_hbm, z1d_hbm,
                     aggp_hbm, updp_hbm,
                     dbuf0, mbuf0, wxb0, wyb0, wzb0, ib00, ib01, ib02, ib03,
                     dbuf1, mbuf1, wxb1, wyb1, wzb1, ib10, ib11, ib12, ib13,
                     onesb, stage, agg_sh, upd_sh,
                     lsem0, lsem1, ssem0, ssem1):
    c = lax.axis_index("c")
    s = lax.axis_index("s")
    wid = s * 2 + c
    r0 = s * RPT
    u0 = s * UPT
    base0 = wid * EPW
    pltpu.sync_copy(z2d_hbm, agg_sh.at[pl.ds(r0, RPT)])
    pltpu.sync_copy(z1d_hbm, stage)
    pltpu.sync_copy(stage, upd_sh.at[pl.ds(u0, UPT)])
    for g in range(GPC):
        onesb[pl.ds(g * 16, 16)] = jnp.full((16,), 1.0, F32)
    plsc.subcore_barrier()

    dbufs = (dbuf0, dbuf1)
    mbufs = (mbuf0, mbuf1)
    wxbs = (wxb0, wxb1)
    wybs = (wyb0, wyb1)
    wzbs = (wzb0, wzb1)
    ibss = ((ib00, ib01, ib02, ib03), (ib10, ib11, ib12, ib13))
    lsems = (lsem0, lsem1)
    ssems = (ssem0, ssem1)

    def fire_loads(jj, b):
        base = base0 + jj * K
        pltpu.async_copy(dst_hbm.at[pl.ds(base, K)], dbufs[b], lsems[b])
        pltpu.async_copy(m2_hbm.at[pl.ds(base, K)], mbufs[b], lsems[b])
        pltpu.async_copy(wx_hbm.at[pl.ds(base, K)], wxbs[b], lsems[b])
        pltpu.async_copy(wy_hbm.at[pl.ds(base, K)], wybs[b], lsems[b])
        pltpu.async_copy(wz_hbm.at[pl.ds(base, K)], wzbs[b], lsems[b])

    def wait_loads(b):
        base = base0
        pltpu.make_async_copy(dst_hbm.at[pl.ds(base, K)], dbufs[b],
                              lsems[b]).wait()
        pltpu.make_async_copy(m2_hbm.at[pl.ds(base, K)], mbufs[b],
                              lsems[b]).wait()
        pltpu.make_async_copy(wx_hbm.at[pl.ds(base, K)], wxbs[b],
                              lsems[b]).wait()
        pltpu.make_async_copy(wy_hbm.at[pl.ds(base, K)], wybs[b],
                              lsems[b]).wait()
        pltpu.make_async_copy(wz_hbm.at[pl.ds(base, K)], wzbs[b],
                              lsems[b]).wait()

    def fire_scatters(b):
        ibs = ibss[b]
        wbs = (wxbs[b], wybs[b], wzbs[b], onesb)
        for g in range(GPC):
            d16 = dbufs[b][pl.ds(g * 16, 16)] * 4
            for jj in range(4):
                ibs[jj][pl.ds(g * 16, 16)] = d16 + jj
        pltpu.async_copy(mbufs[b], agg_sh.at[dbufs[b]], ssems[b], add=True)
        for jj in range(4):
            pltpu.async_copy(wbs[jj], upd_sh.at[ibs[jj]], ssems[b], add=True)

    def wait_scatters(b):
        ibs = ibss[b]
        wbs = (wxbs[b], wybs[b], wzbs[b], onesb)
        pltpu.make_async_copy(mbufs[b], agg_sh.at[dbufs[b]], ssems[b]).wait()
        for jj in range(4):
            pltpu.make_async_copy(wbs[jj], upd_sh.at[ibs[jj]],
                                  ssems[b]).wait()

    fire_loads(0, 0)

    def body(t, carry):
        j0 = t * 2
        wait_loads(0)

        @pl.when(t > 0)
        def _():
            wait_scatters(1)

        fire_loads(j0 + 1, 1)
        fire_scatters(0)
        wait_loads(1)
        wait_scatters(0)

        @pl.when(t < NCHUNK // 2 - 1)
        def _():
            fire_loads(j0 + 2, 0)

        fire_scatters(1)
        return carry

    lax.fori_loop(0, NCHUNK // 2, body, 0)
    wait_scatters(1)
    fire_loads(NCHUNK - 1, 0)
    wait_loads(0)
    fire_scatters(0)
    wait_scatters(0)
    plsc.subcore_barrier()
    pltpu.sync_copy(agg_sh.at[pl.ds(r0, RPT)],
                    aggp_hbm.at[c].at[pl.ds(r0, RPT)])
    pltpu.sync_copy(upd_sh.at[pl.ds(u0, UPT)], stage)
    pltpu.sync_copy(stage, updp_hbm.at[c].at[pl.ds(u0, UPT)])


def _phase4(m2, wx, wy, wz, dst, z2d, z1d):
    mesh = plsc.VectorSubcoreMesh(core_axis_name="c", subcore_axis_name="s")
    kb = [
        pltpu.VMEM((K,), I32),
        pltpu.VMEM((K, D), F32),
        pltpu.VMEM((K,), F32),
        pltpu.VMEM((K,), F32),
        pltpu.VMEM((K,), F32),
        pltpu.VMEM((K,), I32),
        pltpu.VMEM((K,), I32),
        pltpu.VMEM((K,), I32),
        pltpu.VMEM((K,), I32),
    ]
    fn = pl.kernel(
        _sc_scatter_body,
        out_type=(jax.ShapeDtypeStruct((2, NPAD, D), F32),
                  jax.ShapeDtypeStruct((2, UPAD), F32)),
        mesh=mesh,
        scratch_types=kb + kb + [
            pltpu.VMEM((K,), F32),
            pltpu.VMEM((UPT,), F32),
            pltpu.VMEM_SHARED((NPAD, D), F32),
            pltpu.VMEM_SHARED((UPAD,), F32),
            pltpu.SemaphoreType.DMA,
            pltpu.SemaphoreType.DMA,
            pltpu.SemaphoreType.DMA,
            pltpu.SemaphoreType.DMA,
        ],
    )
    return fn(m2, wx, wy, wz, dst, z2d, z1d)


def _p5a_body(x_ref, sig_ref, batch_ref, aggp_ref, updp_ref,
              whx_ref, whc_ref, wha_ref,
              dx_ref, u4_ref, sums_ref):
    i = pl.program_id(0)
    sig = sig_ref[...]
    c_in = lax.rsqrt(SIGMA_DATA * SIGMA_DATA + sig * sig)
    c_noise = jnp.log(sig) * 0.25
    c_skip = (SIGMA_DATA * SIGMA_DATA) * c_in * c_in
    c_out = sig * SIGMA_DATA * c_in
    x = x_ref[...]
    cx = c_in * x
    agg = aggp_ref[0] + aggp_ref[1]
    u8 = updp_ref[0] + updp_ref[1]
    cnt = u8[:, 3:4]
    cnt = jnp.where(cnt == 0.0, 1.0, cnt)
    u3 = u8[:, 0:3] / cnt
    dx = (jnp.dot(cx, whx_ref[...], preferred_element_type=F32)
          + c_noise * whc_ref[...]
          + jnp.dot(agg, wha_ref[...], preferred_element_type=F32))
    dx_ref[...] = c_skip * x + c_out * (cx - dx)
    zero1 = jnp.zeros_like(cnt)
    u4_ref[...] = jnp.concatenate([u3, zero1], axis=1)
    bids = batch_ref[...]
    onehot = (bids == lax.broadcasted_iota(I32, (bids.shape[0], B), 1)
              ).astype(F32)
    u4c = jnp.concatenate([u3, jnp.ones_like(cnt)], axis=1)
    part = lax.dot_general(onehot, u4c, (((0,), (0,)), ((), ())),
                           preferred_element_type=F32)

    @pl.when(i == 0)
    def _():
        sums_ref[...] = part

    @pl.when(i > 0)
    def _():
        sums_ref[...] += part


def _phase5a(x, sig1, batch1, aggp, updp4, whx, whc, wha):
    g = N // NB
    return pl.pallas_call(
        _p5a_body,
        grid=(g,),
        in_specs=[
            pl.BlockSpec((NB, D), lambda i: (i, 0)),
            pl.BlockSpec((NB, 1), lambda i: (i, 0)),
            pl.BlockSpec((NB, 1), lambda i: (i, 0)),
            pl.BlockSpec((2, NB, D), lambda i: (0, i, 0)),
            pl.BlockSpec((2, NB, 4), lambda i: (0, i, 0)),
            pl.BlockSpec((D, D), lambda i: (0, 0)),
            pl.BlockSpec((1, D), lambda i: (0, 0)),
            pl.BlockSpec((D, D), lambda i: (0, 0)),
        ],
        out_specs=[
            pl.BlockSpec((NB, D), lambda i: (i, 0)),
            pl.BlockSpec((NB, 4), lambda i: (i, 0)),
            pl.BlockSpec((B, 4), lambda i: (0, 0)),
        ],
        out_shape=[
            jax.ShapeDtypeStruct((N, D), F32),
            jax.ShapeDtypeStruct((N, 4), F32),
            jax.ShapeDtypeStruct((B, 4), F32),
        ],
    )(x, sig1, batch1, aggp, updp4, whx, whc, wha)


def _p5b_body(rpos4_ref, pos4_ref, sig_ref, batch_ref, u4_ref, sums_ref,
              dpos_ref):
    sig = sig_ref[...]
    c_in = lax.rsqrt(SIGMA_DATA * SIGMA_DATA + sig * sig)
    c_skip = (SIGMA_DATA * SIGMA_DATA) * c_in * c_in
    c_out = sig * SIGMA_DATA * c_in
    sums = sums_ref[...]
    cntb = sums[:, 3:4]
    cntb = jnp.where(cntb == 0.0, 1.0, cntb)
    means4 = sums / cntb
    bids = batch_ref[...]
    onehot = (bids == lax.broadcasted_iota(I32, (bids.shape[0], B), 1)
              ).astype(F32)
    mrow = jnp.dot(onehot, means4, preferred_element_type=F32)
    fpos = pos4_ref[...] + (u4_ref[...] - mrow)
    res = c_skip * rpos4_ref[...] + c_out * fpos
    dpos_ref[...] = res[:, 0:3]


def _phase5b(rpos4, pos4, sig1, batch1, u4, sums):
    g = N // NB
    return pl.pallas_call(
        _p5b_body,
        grid=(g,),
        in_specs=[
            pl.BlockSpec((NB, 4), lambda i: (i, 0)),
            pl.BlockSpec((NB, 4), lambda i: (i, 0)),
            pl.BlockSpec((NB, 1), lambda i: (i, 0)),
            pl.BlockSpec((NB, 1), lambda i: (i, 0)),
            pl.BlockSpec((NB, 4), lambda i: (i, 0)),
            pl.BlockSpec((B, 4), lambda i: (0, 0)),
        ],
        out_specs=pl.BlockSpec((NB, 3), lambda i: (i, 0)),
        out_shape=jax.ShapeDtypeStruct((N, 3), F32),
    )(rpos4, pos4, sig1, batch1, u4, sums)


def kernel(x, pos, edge_index, batch, sigma, We1, We2, Wh, Wx):
    src = edge_index[0]
    dst = edge_index[1]
    sig1 = sigma.reshape(N, 1)
    batch1 = batch.reshape(N, 1)
    pos1 = jnp.concatenate([pos, jnp.ones((N, 1), F32)], axis=1)
    rpos4 = jnp.concatenate([pos, jnp.zeros((N, 1), F32)], axis=1)
    wa = We1[0:D]
    wca = We1[D:D + 1]
    wb = We1[D + 1:2 * D + 1]
    wcb = We1[2 * D + 1:2 * D + 2]
    we1d = We1[2 * D + 2:2 * D + 3]
    whx = Wh[0:D]
    whc = Wh[D:D + 1]
    wha = Wh[D + 1:2 * D + 1]
    wxr = Wx.reshape(1, D)
    psel = jnp.where((jnp.arange(4, dtype=I32)[:, None]
                      == jnp.arange(D, dtype=I32)[None, :])
                     & (jnp.arange(4, dtype=I32)[:, None] < 3),
                     jnp.float32(1.0), jnp.float32(0.0))
    z2d = jnp.zeros((RPT, D), F32)
    z1d = jnp.zeros((UPT,), F32)

    a, b, pp, npp, pos4 = _phase1(x, pos1, sig1, wa, wca, wb, wcb, psel)
    gsum, relp = _phase2(a, b, pp, npp, src, dst)
    m2, wx3, wy3, wz3 = _phase3(gsum, relp, We2, wxr, we1d)
    aggp, updp = _phase4(m2, wx3.reshape(E), wy3.reshape(E), wz3.reshape(E),
                         dst, z2d, z1d)
    aggp = aggp[:, :N]
    updp4 = updp[:, :4 * N].reshape(2, N, 4)
    d_x, u4, sums = _phase5a(x, sig1, batch1, aggp, updp4, whx, whc, wha)
    d_pos = _phase5b(rpos4, pos4, sig1, batch1, u4, sums)
    return (d_x, d_pos)
